# baseline (speedup 1.0000x reference)
import jax
import jax.numpy as jnp
from jax import lax
from jax.experimental import pallas as pl
from jax.experimental.pallas import tpu as pltpu
from jax.experimental.pallas import tpu_sc as plsc

N = 10000
D = 128
E = 320000
NC = 2
NS = 16
NW = NC * NS
EDGES_PER_TILE = E // NW
CHUNK = 128
NCH = 80
PAD = NCH * CHUNK - EDGES_PER_TILE
ROWS_PER_TILE = 624
ROWS_TAIL = N - NS * ROWS_PER_TILE
SUP_ROWS = N + 8


def _mm_body(x_ref, w_ref, o_ref):
    o_ref[pl.ds(0, N), :] = jnp.dot(x_ref[...], w_ref[...],
                                    preferred_element_type=jnp.float32)
    o_ref[pl.ds(N, SUP_ROWS - N), :] = jnp.zeros((SUP_ROWS - N, D),
                                                 jnp.float32)


def _matmul(x, w):
    return pl.pallas_call(
        _mm_body,
        out_shape=jax.ShapeDtypeStruct((SUP_ROWS, w.shape[1]), jnp.float32),
    )(x, w)


def _merge_mm_body(p_ref, b_ref, w_ref, o_ref):
    h = p_ref[0] + p_ref[1] + b_ref[...]
    h = jnp.where(h >= 0, h, 0.25 * h)
    o_ref[pl.ds(0, N), :] = jnp.dot(h, w_ref[...],
                                    preferred_element_type=jnp.float32)
    o_ref[pl.ds(N, SUP_ROWS - N), :] = jnp.zeros((SUP_ROWS - N, D),
                                                 jnp.float32)


def _merge_matmul(partials, b, w):
    return pl.pallas_call(
        _merge_mm_body,
        out_shape=jax.ShapeDtypeStruct((SUP_ROWS, w.shape[1]), jnp.float32),
    )(partials, b, w)


def _merge_act_body(p_ref, b_ref, o_ref):
    h = p_ref[0] + p_ref[1] + b_ref[...]
    o_ref[...] = jnp.where(h >= 0, h, 0.25 * h)


def _merge_act(partials, b):
    return pl.pallas_call(
        _merge_act_body,
        out_shape=jax.ShapeDtypeStruct((N, D), jnp.float32),
    )(partials, b)


def _sc_scatter_body(sup_hbm, rowp_hbm, colp_hbm, zero_hbm, out_hbm,
                     colv, rowv, gat, acc):
    cid = lax.axis_index("c")
    sid = lax.axis_index("s")
    wid = cid * NS + sid
    rbase = sid * ROWS_PER_TILE

    pltpu.sync_copy(zero_hbm.at[pl.ds(rbase, ROWS_PER_TILE)],
                    acc.at[pl.ds(rbase, ROWS_PER_TILE)])

    @pl.when(sid == NS - 1)
    def _():
        pltpu.sync_copy(zero_hbm.at[pl.ds(NS * ROWS_PER_TILE, ROWS_TAIL)],
                        acc.at[pl.ds(NS * ROWS_PER_TILE, ROWS_TAIL)])

    plsc.subcore_barrier()

    ebase = wid * (NCH * CHUNK)

    @pl.loop(0, NCH)
    def _(j):
        base = ebase + j * CHUNK
        pltpu.sync_copy(colp_hbm.at[pl.ds(base, CHUNK)], colv)
        pltpu.sync_copy(rowp_hbm.at[pl.ds(base, CHUNK)], rowv)
        pltpu.sync_copy(sup_hbm.at[colv], gat)
        pltpu.sync_copy(gat, acc.at[rowv], add=True)

    plsc.subcore_barrier()
    pltpu.sync_copy(acc.at[pl.ds(rbase, ROWS_PER_TILE)],
                    out_hbm.at[cid, pl.ds(rbase, ROWS_PER_TILE)])

    @pl.when(sid == NS - 1)
    def _():
        pltpu.sync_copy(acc.at[pl.ds(NS * ROWS_PER_TILE, ROWS_TAIL)],
                        out_hbm.at[cid, pl.ds(NS * ROWS_PER_TILE, ROWS_TAIL)])


def _sc_scatter_add(support, rowp, colp, zeros):
    mesh = plsc.VectorSubcoreMesh(core_axis_name="c", subcore_axis_name="s")
    k = pl.kernel(
        _sc_scatter_body,
        out_type=jax.ShapeDtypeStruct((NC, N, D), jnp.float32),
        mesh=mesh,
        scratch_types=[
            pltpu.VMEM((CHUNK,), jnp.int32),
            pltpu.VMEM((CHUNK,), jnp.int32),
            pltpu.VMEM((CHUNK, D), jnp.float32),
            pltpu.VMEM_SHARED((N, D), jnp.float32),
        ],
    )
    return k(support, rowp, colp, zeros)


def kernel(x, edge_index, W1, b1, W2, b2):
    ei = edge_index.astype(jnp.int32)
    rowp = jnp.pad(ei[0].reshape(NW, EDGES_PER_TILE), ((0, 0), (0, PAD)),
                   constant_values=0).reshape(NW * NCH * CHUNK)
    colp = jnp.pad(ei[1].reshape(NW, EDGES_PER_TILE), ((0, 0), (0, PAD)),
                   constant_values=N).reshape(NW * NCH * CHUNK)
    zeros = jnp.zeros((N, D), jnp.float32)
    b1r = jnp.reshape(b1, (1, D))
    b2r = jnp.reshape(b2, (1, D))

    support1 = _matmul(x, W1)
    part1 = _sc_scatter_add(support1, rowp, colp, zeros)
    support2 = _merge_matmul(part1, b1r, W2)
    part2 = _sc_scatter_add(support2, rowp, colp, zeros)
    return _merge_act(part2, b2r)


# speedup vs baseline: 2.0380x; 2.0380x over previous
import jax
import jax.numpy as jnp
from jax import lax
from jax.experimental import pallas as pl
from jax.experimental.pallas import tpu as pltpu
from jax.experimental.pallas import tpu_sc as plsc

N = 10000
D = 128
E = 320000
NC = 2
NS = 16
EDGES_PER_SC = E // NC
EDGES_PER_TILE = EDGES_PER_SC // NS
CHUNK = 128
NFULL = EDGES_PER_TILE // CHUNK
REM = EDGES_PER_TILE - NFULL * CHUNK
ROWS_PER_TILE = 624
ROWS_TAIL = N - NS * ROWS_PER_TILE


def _mm_body(x_ref, w_ref, o_ref):
    o_ref[...] = jnp.dot(x_ref[...], w_ref[...],
                         preferred_element_type=jnp.float32)


def _matmul(x, w):
    return pl.pallas_call(
        _mm_body,
        out_shape=jax.ShapeDtypeStruct((x.shape[0], w.shape[1]), jnp.float32),
    )(x, w)


def _merge_mm_body(p_ref, b_ref, w_ref, o_ref):
    h = p_ref[0] + p_ref[1] + b_ref[...]
    h = jnp.where(h >= 0, h, 0.25 * h)
    o_ref[...] = jnp.dot(h, w_ref[...], preferred_element_type=jnp.float32)


def _merge_matmul(partials, b, w):
    return pl.pallas_call(
        _merge_mm_body,
        out_shape=jax.ShapeDtypeStruct((N, w.shape[1]), jnp.float32),
    )(partials, b, w)


def _merge_act_body(p_ref, b_ref, o_ref):
    h = p_ref[0] + p_ref[1] + b_ref[...]
    o_ref[...] = jnp.where(h >= 0, h, 0.25 * h)


def _merge_act(partials, b):
    return pl.pallas_call(
        _merge_act_body,
        out_shape=jax.ShapeDtypeStruct((N, D), jnp.float32),
    )(partials, b)


def _sc_scatter_body(sup_hbm, row_hbm, col_hbm, zero_hbm, out_hbm,
                     colv, rowv, gat, colr, rowr, gatr, acc):
    cid = lax.axis_index("c")
    sid = lax.axis_index("s")
    rbase = sid * ROWS_PER_TILE
    pltpu.sync_copy(zero_hbm.at[pl.ds(rbase, ROWS_PER_TILE)],
                    acc.at[pl.ds(rbase, ROWS_PER_TILE)])

    @pl.when(sid == NS - 1)
    def _():
        pltpu.sync_copy(zero_hbm.at[pl.ds(NS * ROWS_PER_TILE, ROWS_TAIL)],
                        acc.at[pl.ds(NS * ROWS_PER_TILE, ROWS_TAIL)])

    plsc.subcore_barrier()

    ebase = cid * EDGES_PER_SC + sid * EDGES_PER_TILE

    @pl.loop(0, NFULL)
    def _(i):
        base = ebase + i * CHUNK
        pltpu.sync_copy(col_hbm.at[pl.ds(base, CHUNK)], colv)
        pltpu.sync_copy(row_hbm.at[pl.ds(base, CHUNK)], rowv)
        pltpu.sync_copy(sup_hbm.at[colv], gat)
        pltpu.sync_copy(gat, acc.at[rowv], add=True)

    base = ebase + NFULL * CHUNK
    pltpu.sync_copy(col_hbm.at[pl.ds(base, REM)], colr)
    pltpu.sync_copy(row_hbm.at[pl.ds(base, REM)], rowr)
    pltpu.sync_copy(sup_hbm.at[colr], gatr)
    pltpu.sync_copy(gatr, acc.at[rowr], add=True)

    plsc.subcore_barrier()
    pltpu.sync_copy(acc.at[pl.ds(rbase, ROWS_PER_TILE)],
                    out_hbm.at[cid, pl.ds(rbase, ROWS_PER_TILE)])

    @pl.when(sid == NS - 1)
    def _():
        pltpu.sync_copy(acc.at[pl.ds(NS * ROWS_PER_TILE, ROWS_TAIL)],
                        out_hbm.at[cid, pl.ds(NS * ROWS_PER_TILE, ROWS_TAIL)])


def _sc_scatter_add(support, row, col, zeros):
    mesh = plsc.VectorSubcoreMesh(core_axis_name="c", subcore_axis_name="s")
    k = pl.kernel(
        _sc_scatter_body,
        out_type=jax.ShapeDtypeStruct((NC, N, D), jnp.float32),
        mesh=mesh,
        scratch_types=[
            pltpu.VMEM((CHUNK,), jnp.int32),
            pltpu.VMEM((CHUNK,), jnp.int32),
            pltpu.VMEM((CHUNK, D), jnp.float32),
            pltpu.VMEM((REM,), jnp.int32),
            pltpu.VMEM((REM,), jnp.int32),
            pltpu.VMEM((REM, D), jnp.float32),
            pltpu.VMEM_SHARED((N, D), jnp.float32),
        ],
    )
    return k(support, row, col, zeros)


def kernel(x, edge_index, W1, b1, W2, b2):
    ei = edge_index.astype(jnp.int32)
    row = ei[0]
    col = ei[1]
    zeros = jnp.zeros((N, D), jnp.float32)
    b1r = jnp.reshape(b1, (1, D))
    b2r = jnp.reshape(b2, (1, D))

    support1 = _matmul(x, W1)
    part1 = _sc_scatter_add(support1, row, col, zeros)
    support2 = _merge_matmul(part1, b1r, W2)
    part2 = _sc_scatter_add(support2, row, col, zeros)
    return _merge_act(part2, b2r)


# speedup vs baseline: 2.6359x; 1.2934x over previous
import jax
import jax.numpy as jnp
from jax import lax
from jax.experimental import pallas as pl
from jax.experimental.pallas import tpu as pltpu
from jax.experimental.pallas import tpu_sc as plsc

N = 10000
D = 128
E = 320000
NC = 2
NS = 16
EDGES_PER_SC = E // NC
EDGES_PER_TILE = EDGES_PER_SC // NS
NW = NC * NS
CHUNK = 128
NCH = 80
PAD = NCH * CHUNK - EDGES_PER_TILE
TRASH = 64
ROWS_PER_TILE = 624
ROWS_TAIL = N - NS * ROWS_PER_TILE


def _mm_body(x_ref, w_ref, o_ref):
    o_ref[...] = jnp.dot(x_ref[...], w_ref[...],
                         preferred_element_type=jnp.float32)


def _matmul(x, w):
    return pl.pallas_call(
        _mm_body,
        out_shape=jax.ShapeDtypeStruct((x.shape[0], w.shape[1]), jnp.float32),
    )(x, w)


def _merge_mm_body(p_ref, b_ref, w_ref, o_ref):
    h = p_ref[0] + p_ref[1] + b_ref[...]
    h = jnp.where(h >= 0, h, 0.25 * h)
    o_ref[...] = jnp.dot(h, w_ref[...], preferred_element_type=jnp.float32)


def _merge_matmul(partials, b, w):
    return pl.pallas_call(
        _merge_mm_body,
        out_shape=jax.ShapeDtypeStruct((N, w.shape[1]), jnp.float32),
    )(partials, b, w)


def _merge_act_body(p_ref, b_ref, o_ref):
    h = p_ref[0] + p_ref[1] + b_ref[...]
    o_ref[...] = jnp.where(h >= 0, h, 0.25 * h)


def _merge_act(partials, b):
    return pl.pallas_call(
        _merge_act_body,
        out_shape=jax.ShapeDtypeStruct((N, D), jnp.float32),
    )(partials, b)


def _sc_scatter_body(sup_hbm, row_hbm, col_hbm, zero_hbm, out_hbm,
                     colv, rowv, gat, acc):
    cid = lax.axis_index("c")
    sid = lax.axis_index("s")
    wid = cid * NS + sid
    rbase = sid * ROWS_PER_TILE
    pltpu.sync_copy(zero_hbm.at[pl.ds(rbase, ROWS_PER_TILE)],
                    acc.at[pl.ds(rbase, ROWS_PER_TILE)])

    @pl.when(sid == NS - 1)
    def _():
        pltpu.sync_copy(zero_hbm.at[pl.ds(NS * ROWS_PER_TILE, ROWS_TAIL)],
                        acc.at[pl.ds(NS * ROWS_PER_TILE, ROWS_TAIL)])

    plsc.subcore_barrier()

    pltpu.sync_copy(col_hbm.at[wid], colv)
    pltpu.sync_copy(row_hbm.at[wid], rowv)

    for j in range(NCH):
        pltpu.sync_copy(sup_hbm.at[colv.at[j]], gat)
        pltpu.sync_copy(gat, acc.at[rowv.at[j]], add=True)

    plsc.subcore_barrier()
    pltpu.sync_copy(acc.at[pl.ds(rbase, ROWS_PER_TILE)],
                    out_hbm.at[cid, pl.ds(rbase, ROWS_PER_TILE)])

    @pl.when(sid == NS - 1)
    def _():
        pltpu.sync_copy(acc.at[pl.ds(NS * ROWS_PER_TILE, ROWS_TAIL)],
                        out_hbm.at[cid, pl.ds(NS * ROWS_PER_TILE, ROWS_TAIL)])


def _sc_scatter_add(support, row, col, zeros):
    mesh = plsc.VectorSubcoreMesh(core_axis_name="c", subcore_axis_name="s")
    k = pl.kernel(
        _sc_scatter_body,
        out_type=jax.ShapeDtypeStruct((NC, N, D), jnp.float32),
        mesh=mesh,
        scratch_types=[
            pltpu.VMEM((NCH, CHUNK), jnp.int32),
            pltpu.VMEM((NCH, CHUNK), jnp.int32),
            pltpu.VMEM((CHUNK, D), jnp.float32),
            pltpu.VMEM_SHARED((N + TRASH, D), jnp.float32),
        ],
    )
    return k(support, row, col, zeros)


def kernel(x, edge_index, W1, b1, W2, b2):
    ei = edge_index.astype(jnp.int32)
    padix = jnp.arange(PAD, dtype=jnp.int32)
    rpad = jnp.broadcast_to(N + (padix % TRASH), (NW, PAD))
    cpad = jnp.broadcast_to((padix * 41) % N, (NW, PAD))
    row = jnp.concatenate([ei[0].reshape(NW, EDGES_PER_TILE), rpad],
                          axis=1).reshape(NW, NCH, CHUNK)
    col = jnp.concatenate([ei[1].reshape(NW, EDGES_PER_TILE), cpad],
                          axis=1).reshape(NW, NCH, CHUNK)
    zeros = jnp.zeros((N, D), jnp.float32)
    b1r = jnp.reshape(b1, (1, D))
    b2r = jnp.reshape(b2, (1, D))

    support1 = _matmul(x, W1)
    part1 = _sc_scatter_add(support1, row, col, zeros)
    support2 = _merge_matmul(part1, b1r, W2)
    part2 = _sc_scatter_add(support2, row, col, zeros)
    return _merge_act(part2, b2r)
